# baseline (speedup 1.0000x reference)
import functools

import jax
import jax.numpy as jnp
from jax import lax
from jax.experimental import pallas as pl
from jax.experimental.pallas import tpu as pltpu
from jax.experimental.pallas import tpu_sc as plsc

N = 10000
D = 128
E = 320000

NC = 2
NS = 16
NW = NC * NS
K = 128
IB = 8
NIT = 80
EPT = NIT * K
EPAD = EPT * NW
NBLKS = NIT // IB
NP = 10240
RPT = NP // NS

R = 2000
NBLK = N // R


def _sc_mesh():
    return plsc.VectorSubcoreMesh(core_axis_name="c", subcore_axis_name="s")


@functools.partial(
    pl.kernel,
    mesh=_sc_mesh(),
    out_type=jax.ShapeDtypeStruct((2, NP), jnp.float32),
    scratch_types=[
        pltpu.VMEM((K,), jnp.float32),
        pltpu.VMEM((RPT,), jnp.float32),
        pltpu.VMEM((IB, K), jnp.int32),
        pltpu.VMEM_SHARED((NP,), jnp.float32),
        pltpu.SemaphoreType.DMA,
    ],
)
def _deg_kernel(dst2d_hbm, out_hbm, ones_v, zero_v, idx_v, acc_sh, sem):
    cid = lax.axis_index("c")
    sid = lax.axis_index("s")
    wid = cid * NS + sid
    for j in range(K // 16):
        ones_v[pl.ds(j * 16, 16)] = jnp.full((16,), 1.0, jnp.float32)
    for j in range(RPT // 16):
        zero_v[pl.ds(j * 16, 16)] = jnp.zeros((16,), jnp.float32)
    pltpu.sync_copy(zero_v, acc_sh.at[pl.ds(sid * RPT, RPT)])
    plsc.subcore_barrier()

    @pl.loop(0, NBLKS)
    def _blk(g):
        row0 = pl.multiple_of(wid * NIT + g * IB, IB)
        pltpu.sync_copy(dst2d_hbm.at[pl.ds(row0, IB)], idx_v)
        descs = [
            pltpu.async_copy(ones_v, acc_sh.at[idx_v.at[j]], sem, add=True)
            for j in range(IB)
        ]
        for d in descs:
            d.wait()

    plsc.subcore_barrier()
    pltpu.sync_copy(acc_sh.at[pl.ds(sid * RPT, RPT)],
                    out_hbm.at[cid, pl.ds(sid * RPT, RPT)])


@functools.partial(
    pl.kernel,
    mesh=_sc_mesh(),
    out_type=jax.ShapeDtypeStruct((2, NP, D), jnp.float32),
    scratch_types=[
        pltpu.VMEM((K, D), jnp.float32),
        pltpu.VMEM((K, D), jnp.float32),
        pltpu.VMEM((IB, K), jnp.int32),
        pltpu.VMEM((IB, K), jnp.int32),
        pltpu.VMEM_SHARED((NP, D), jnp.float32),
        pltpu.SemaphoreType.DMA,
        pltpu.SemaphoreType.DMA,
        pltpu.SemaphoreType.DMA,
        pltpu.SemaphoreType.DMA,
    ],
)
def _msg_kernel(src2d_hbm, dst2d_hbm, h_hbm, zeros_hbm, out_hbm,
                rows0, rows1, isrc_v, idst_v, acc_sh,
                gsem0, gsem1, ssem0, ssem1):
    cid = lax.axis_index("c")
    sid = lax.axis_index("s")
    wid = cid * NS + sid
    rows = (rows0, rows1)
    gsem = (gsem0, gsem1)
    ssem = (ssem0, ssem1)
    pltpu.sync_copy(zeros_hbm, acc_sh.at[pl.ds(sid * RPT, RPT)])
    plsc.subcore_barrier()

    @pl.loop(0, NBLKS)
    def _blk(g):
        row0 = pl.multiple_of(wid * NIT + g * IB, IB)
        pltpu.sync_copy(src2d_hbm.at[pl.ds(row0, IB)], isrc_v)
        pltpu.sync_copy(dst2d_hbm.at[pl.ds(row0, IB)], idst_v)
        gd = [None, None]
        sd = [None, None]
        gd[0] = pltpu.async_copy(h_hbm.at[isrc_v.at[0]], rows[0], gsem[0])
        for j in range(1, IB):
            b = j % 2
            pb = 1 - b
            if sd[b] is not None:
                sd[b].wait()
            gd[b] = pltpu.async_copy(h_hbm.at[isrc_v.at[j]], rows[b], gsem[b])
            gd[pb].wait()
            sd[pb] = pltpu.async_copy(rows[pb], acc_sh.at[idst_v.at[j - 1]],
                                      ssem[pb], add=True)
        last = (IB - 1) % 2
        gd[last].wait()
        sd[last] = pltpu.async_copy(rows[last], acc_sh.at[idst_v.at[IB - 1]],
                                    ssem[last], add=True)
        sd[0].wait()
        sd[1].wait()

    plsc.subcore_barrier()
    pltpu.sync_copy(acc_sh.at[pl.ds(sid * RPT, RPT)],
                    out_hbm.at[cid, pl.ds(sid * RPT, RPT)])


def _h_body(feat_ref, dga_ref, dgb_ref, h_ref):
    d = dga_ref[0] + dgb_ref[0]
    norm = lax.rsqrt(jnp.maximum(d, 1.0))
    h_ref[...] = feat_ref[...] * norm


def _h_call(feat, deg2):
    return pl.pallas_call(
        _h_body,
        grid=(NBLK,),
        in_specs=[
            pl.BlockSpec((R, D), lambda i: (i, 0)),
            pl.BlockSpec((1, R, 1), lambda i: (0, i, 0)),
            pl.BlockSpec((1, R, 1), lambda i: (1, i, 0)),
        ],
        out_specs=pl.BlockSpec((R, D), lambda i: (i, 0)),
        out_shape=jax.ShapeDtypeStruct((N, D), jnp.float32),
    )(feat, deg2, deg2)


def _out_body(msga_ref, msgb_ref, h_ref, dga_ref, dgb_ref, lin_ref,
              w_ref, b_ref, o_ref):
    msg = msga_ref[0] + msgb_ref[0]
    d = dga_ref[0] + dgb_ref[0]
    norm = lax.rsqrt(jnp.maximum(d, 1.0))
    l = lin_ref[...]
    out = (1.0 - l) * msg + l * h_ref[...]
    r = jnp.dot(out, w_ref[...], preferred_element_type=jnp.float32)
    o_ref[...] = r * norm + b_ref[...]


def _out_call(msg2, h, deg2, lin, weight, bias):
    return pl.pallas_call(
        _out_body,
        grid=(NBLK,),
        in_specs=[
            pl.BlockSpec((1, R, D), lambda i: (0, i, 0)),
            pl.BlockSpec((1, R, D), lambda i: (1, i, 0)),
            pl.BlockSpec((R, D), lambda i: (i, 0)),
            pl.BlockSpec((1, R, 1), lambda i: (0, i, 0)),
            pl.BlockSpec((1, R, 1), lambda i: (1, i, 0)),
            pl.BlockSpec((R, 1), lambda i: (i, 0)),
            pl.BlockSpec((D, D), lambda i: (0, 0)),
            pl.BlockSpec((1, D), lambda i: (0, 0)),
        ],
        out_specs=pl.BlockSpec((R, D), lambda i: (i, 0)),
        out_shape=jax.ShapeDtypeStruct((N, D), jnp.float32),
    )(msg2, msg2, h, deg2, deg2, lin, weight, bias)


def kernel(feat, edge_index, weight, bias, linear_comb):
    pad = EPAD - E
    src = jnp.concatenate([edge_index[0], jnp.zeros((pad,), jnp.int32)])
    dst_pad = N + jnp.arange(pad, dtype=jnp.int32) % (NP - N)
    dst = jnp.concatenate([edge_index[1], dst_pad])
    src2d = src.reshape(EPAD // K, K)
    dst2d = dst.reshape(EPAD // K, K)
    zeros128 = jnp.zeros((RPT, D), jnp.float32)

    deg2 = _deg_kernel(dst2d).reshape(2, NP, 1)
    h = _h_call(feat, deg2)
    msg2 = _msg_kernel(src2d, dst2d, h, zeros128)
    rst = _out_call(msg2, h, deg2, linear_comb.reshape(N, 1),
                    weight, bias.reshape(1, D))
    return rst


# speedup vs baseline: 1.2583x; 1.2583x over previous
import functools

import jax
import jax.numpy as jnp
from jax import lax
from jax.experimental import pallas as pl
from jax.experimental.pallas import tpu as pltpu
from jax.experimental.pallas import tpu_sc as plsc

N = 10000
D = 128
E = 320000

NC = 2
NS = 16
NW = NC * NS
K = 128
EPT = -(-E // (NW * K)) * K
EPAD = EPT * NW
NIT = EPT // K
NP = 10240
RPT = NP // NS
DW = 16

IB = 8
NITD = 80
EPADD = NITD * K * NW
NBLKSD = NITD // IB

R = 2000
NBLK = N // R


def _sc_mesh():
    return plsc.VectorSubcoreMesh(core_axis_name="c", subcore_axis_name="s")


@functools.partial(
    pl.kernel,
    mesh=_sc_mesh(),
    out_type=jax.ShapeDtypeStruct((2, NP), jnp.float32),
    scratch_types=[
        pltpu.VMEM((K,), jnp.float32),
        pltpu.VMEM((RPT,), jnp.float32),
        pltpu.VMEM((IB, K), jnp.int32),
        pltpu.VMEM_SHARED((NP,), jnp.float32),
        pltpu.SemaphoreType.DMA,
    ],
)
def _deg_kernel(dst2d_hbm, out_hbm, ones_v, zero_v, idx_v, acc_sh, sem):
    cid = lax.axis_index("c")
    sid = lax.axis_index("s")
    wid = cid * NS + sid
    for j in range(K // 16):
        ones_v[pl.ds(j * 16, 16)] = jnp.full((16,), 1.0, jnp.float32)
    for j in range(RPT // 16):
        zero_v[pl.ds(j * 16, 16)] = jnp.zeros((16,), jnp.float32)
    pltpu.sync_copy(zero_v, acc_sh.at[pl.ds(sid * RPT, RPT)])
    plsc.subcore_barrier()

    @pl.loop(0, NBLKSD)
    def _blk(g):
        row0 = pl.multiple_of(wid * NITD + g * IB, IB)
        pltpu.sync_copy(dst2d_hbm.at[pl.ds(row0, IB)], idx_v)
        descs = [
            pltpu.async_copy(ones_v, acc_sh.at[idx_v.at[j]], sem, add=True)
            for j in range(IB)
        ]
        for d in descs:
            d.wait()

    plsc.subcore_barrier()
    pltpu.sync_copy(acc_sh.at[pl.ds(sid * RPT, RPT)],
                    out_hbm.at[cid, pl.ds(sid * RPT, RPT)])


@functools.partial(
    pl.kernel,
    mesh=_sc_mesh(),
    out_type=jax.ShapeDtypeStruct((2, NP, D), jnp.float32),
    scratch_types=[
        pltpu.VMEM((K, D), jnp.float32),
        pltpu.VMEM((K, D), jnp.float32),
        pltpu.VMEM((K,), jnp.int32),
        pltpu.VMEM((K,), jnp.int32),
        pltpu.VMEM((K,), jnp.int32),
        pltpu.VMEM((K,), jnp.int32),
        pltpu.VMEM_SHARED((NP, D), jnp.float32),
        pltpu.SemaphoreType.DMA,
        pltpu.SemaphoreType.DMA,
    ],
)
def _msg_kernel(src_hbm, dst_hbm, h_hbm, zeros_hbm, out_hbm,
                rows0, rows1, isrc0, idst0, isrc1, idst1, acc_sh,
                gsem0, gsem1):
    cid = lax.axis_index("c")
    sid = lax.axis_index("s")
    wid = cid * NS + sid
    pltpu.sync_copy(zeros_hbm, acc_sh.at[pl.ds(sid * RPT, RPT)])
    plsc.subcore_barrier()

    def body(g, carry):
        base0 = pl.multiple_of(wid * EPT + (2 * g) * K, 8)
        base1 = pl.multiple_of(wid * EPT + (2 * g + 1) * K, 8)
        pltpu.sync_copy(src_hbm.at[pl.ds(base0, K)], isrc0)
        pltpu.sync_copy(dst_hbm.at[pl.ds(base0, K)], idst0)
        gd0 = pltpu.async_copy(h_hbm.at[isrc0], rows0, gsem0)
        pltpu.sync_copy(src_hbm.at[pl.ds(base1, K)], isrc1)
        pltpu.sync_copy(dst_hbm.at[pl.ds(base1, K)], idst1)
        gd1 = pltpu.async_copy(h_hbm.at[isrc1], rows1, gsem1)
        gd0.wait()
        pltpu.sync_copy(rows0, acc_sh.at[idst0], add=True)
        gd1.wait()
        pltpu.sync_copy(rows1, acc_sh.at[idst1], add=True)
        return carry

    lax.fori_loop(0, NIT // 2, body, 0)
    tbase = pl.multiple_of(wid * EPT + (NIT - 1) * K, 8)
    pltpu.sync_copy(src_hbm.at[pl.ds(tbase, K)], isrc0)
    pltpu.sync_copy(dst_hbm.at[pl.ds(tbase, K)], idst0)
    pltpu.async_copy(h_hbm.at[isrc0], rows0, gsem0).wait()
    pltpu.sync_copy(rows0, acc_sh.at[idst0], add=True)
    plsc.subcore_barrier()
    pltpu.sync_copy(acc_sh.at[pl.ds(sid * RPT, RPT)],
                    out_hbm.at[cid, pl.ds(sid * RPT, RPT)])


def _h_body(feat_ref, dga_ref, dgb_ref, h_ref):
    d = dga_ref[0] + dgb_ref[0]
    norm = lax.rsqrt(jnp.maximum(d, 1.0))
    h_ref[...] = feat_ref[...] * norm


def _h_call(feat, deg2):
    return pl.pallas_call(
        _h_body,
        grid=(NBLK,),
        in_specs=[
            pl.BlockSpec((R, D), lambda i: (i, 0)),
            pl.BlockSpec((1, R, 1), lambda i: (0, i, 0)),
            pl.BlockSpec((1, R, 1), lambda i: (1, i, 0)),
        ],
        out_specs=pl.BlockSpec((R, D), lambda i: (i, 0)),
        out_shape=jax.ShapeDtypeStruct((N, D), jnp.float32),
    )(feat, deg2, deg2)


def _out_body(msga_ref, msgb_ref, h_ref, dga_ref, dgb_ref, lin_ref,
              w_ref, b_ref, o_ref):
    msg = msga_ref[0] + msgb_ref[0]
    d = dga_ref[0] + dgb_ref[0]
    norm = lax.rsqrt(jnp.maximum(d, 1.0))
    l = lin_ref[...]
    out = (1.0 - l) * msg + l * h_ref[...]
    r = jnp.dot(out, w_ref[...], preferred_element_type=jnp.float32)
    o_ref[...] = r * norm + b_ref[...]


def _out_call(msg2, h, deg2, lin, weight, bias):
    return pl.pallas_call(
        _out_body,
        grid=(NBLK,),
        in_specs=[
            pl.BlockSpec((1, R, D), lambda i: (0, i, 0)),
            pl.BlockSpec((1, R, D), lambda i: (1, i, 0)),
            pl.BlockSpec((R, D), lambda i: (i, 0)),
            pl.BlockSpec((1, R, 1), lambda i: (0, i, 0)),
            pl.BlockSpec((1, R, 1), lambda i: (1, i, 0)),
            pl.BlockSpec((R, 1), lambda i: (i, 0)),
            pl.BlockSpec((D, D), lambda i: (0, 0)),
            pl.BlockSpec((1, D), lambda i: (0, 0)),
        ],
        out_specs=pl.BlockSpec((R, D), lambda i: (i, 0)),
        out_shape=jax.ShapeDtypeStruct((N, D), jnp.float32),
    )(msg2, msg2, h, deg2, deg2, lin, weight, bias)


def kernel(feat, edge_index, weight, bias, linear_comb):
    pad = EPAD - E
    src = jnp.concatenate([edge_index[0], jnp.zeros((pad,), jnp.int32)])
    dst = jnp.concatenate([edge_index[1], jnp.full((pad,), N, jnp.int32)])
    padd = EPADD - E
    dstd_pad = N + jnp.arange(padd, dtype=jnp.int32) % (NP - N)
    dstd2d = jnp.concatenate([edge_index[1], dstd_pad]).reshape(EPADD // K, K)
    zeros128 = jnp.zeros((RPT, D), jnp.float32)

    deg2 = _deg_kernel(dstd2d).reshape(2, NP, 1)
    h = _h_call(feat, deg2)
    msg2 = _msg_kernel(src, dst, h, zeros128)
    rst = _out_call(msg2, h, deg2, linear_comb.reshape(N, 1),
                    weight, bias.reshape(1, D))
    return rst
